# baseline (device time: 33707 ns/iter reference)
import jax
import jax.numpy as jnp
from jax import lax
from jax.experimental import pallas as pl
from jax.experimental.pallas import tpu as pltpu

N_DEV = 8
SQ = 256
D = 1024
DH = 128
H_PER = 8
SCALE = 0.08838834764831843

RAIL_MASKS = ((1, 3, 4), (3, 4, 1), (4, 1, 3))
RAIL_COLS = ((0, 384), (384, 768), (768, 1024))
STAGE_OFF = (0, 128, 192)


def kernel(x, Wq, Wo, Wk, Wv):
    def body(x_ref, wq_ref, wo_ref, wk_ref, wv_ref, out_ref,
             acc_ref, rs_recv, send_sems, recv_sems):
        my = lax.axis_index("i")
        b0 = my & 1
        b1 = (my >> 1) & 1
        b2 = (my >> 2) & 1
        sels = (
            (b0 ^ b1, b1, b2),
            (b1, b2, b0),
            (b2, b0 ^ b1, b1),
        )

        barrier = pltpu.get_barrier_semaphore()
        for msk in (1, 3, 4):
            pl.semaphore_signal(barrier, inc=1, device_id=(my ^ msk,),
                                device_id_type=pl.DeviceIdType.MESH)
        pl.semaphore_wait(barrier, 3)

        plans = []
        for r, ((m0, m1, m2), (s0, s1, s2)) in enumerate(
                zip(RAIL_MASKS, sels)):
            k0 = s0 * 128
            k1 = k0 + s1 * 64
            k2 = k1 + s2 * 32
            c0, c1 = RAIL_COLS[r]
            plans.append({
                "cols": (c0, c1),
                "s0": s0,
                "rs": [
                    (m0, (1 - s0) * 128, 128, k0),
                    (m1, k0 + (1 - s1) * 64, 64, k1),
                    (m2, k1 + (1 - s2) * 32, 32, k2),
                ],
                "ag": [(m2, k2, 32), (m1, k1, 64), (m0, k0, 128)],
            })

        def rs_rdma(r, s):
            msk, src_row, sz, _ = plans[r]["rs"][s]
            c0, c1 = plans[r]["cols"]
            return pltpu.make_async_remote_copy(
                src_ref=acc_ref.at[pl.ds(src_row, sz), c0:c1],
                dst_ref=rs_recv.at[pl.ds(STAGE_OFF[s], sz), c0:c1],
                send_sem=send_sems.at[r * 6 + s],
                recv_sem=recv_sems.at[r * 6 + s],
                device_id=(my ^ msk,),
                device_id_type=pl.DeviceIdType.MESH,
            )

        def ag_rdma(r, s):
            msk, row, sz = plans[r]["ag"][s]
            c0, c1 = plans[r]["cols"]
            return pltpu.make_async_remote_copy(
                src_ref=acc_ref.at[pl.ds(row, sz), c0:c1],
                dst_ref=acc_ref.at[pl.ds(row, sz), c0:c1],
                send_sem=send_sems.at[r * 6 + 3 + s],
                recv_sem=recv_sems.at[r * 6 + 3 + s],
                device_id=(my ^ msk,),
                device_id_type=pl.DeviceIdType.MESH,
            )

        xb = x_ref[0].astype(jnp.bfloat16)
        wob = wo_ref[...].astype(jnp.bfloat16)
        bf = jnp.bfloat16
        k = jnp.dot(xb, wk_ref[...].astype(bf),
                    preferred_element_type=jnp.float32).astype(bf)
        v = jnp.dot(xb, wv_ref[...].astype(bf),
                    preferred_element_type=jnp.float32).astype(bf)
        q = jnp.dot(xb, wq_ref[...].astype(bf),
                    preferred_element_type=jnp.float32).astype(bf)

        def compute_rows(r0):
            outs = []
            for h in range(H_PER):
                sl = slice(h * DH, (h + 1) * DH)
                s = jnp.dot(q[r0:r0 + 128, sl], k[:, sl].T,
                            preferred_element_type=jnp.float32) * SCALE
                m = jnp.max(s, axis=-1, keepdims=True)
                p = jnp.exp(s - m)
                l = jnp.sum(p, axis=-1, keepdims=True)
                outs.append(jnp.dot((p / l).astype(bf), v[:, sl],
                                    preferred_element_type=jnp.float32))
            attn = jnp.concatenate(outs, axis=1).astype(bf)
            part = jnp.dot(attn, wob,
                           preferred_element_type=jnp.float32)
            acc_ref[r0:r0 + 128, :] = part.astype(jnp.bfloat16)

        compute_rows(128)
        for r in range(3):
            @pl.when(plans[r]["s0"] == 0)
            def _(r=r):
                rs_rdma(r, 0).start()
        compute_rows(0)
        for r in range(3):
            @pl.when(plans[r]["s0"] == 1)
            def _(r=r):
                rs_rdma(r, 0).start()

        for s in range(3):
            for r in range(3):
                rs_rdma(r, s).wait()
                _, _, sz, add_row = plans[r]["rs"][s]
                c0, c1 = plans[r]["cols"]
                if s < 2:
                    nxt = rs_rdma(r, s + 1)
                else:
                    nxt = ag_rdma(r, 0)
                acc_ref[pl.ds(add_row, sz), c0:c1] += (
                    rs_recv[pl.ds(STAGE_OFF[s], sz), c0:c1])
                nxt.start()
        for s in range(3):
            for r in range(3):
                ag_rdma(r, s).wait()
                if s < 2:
                    ag_rdma(r, s + 1).start()

        out_ref[0] = acc_ref[...].astype(jnp.float32)

    return pl.pallas_call(
        body,
        out_shape=jax.ShapeDtypeStruct((1, SQ, D), jnp.float32),
        in_specs=[pl.BlockSpec(memory_space=pltpu.VMEM)] * 5,
        out_specs=pl.BlockSpec(memory_space=pltpu.VMEM),
        scratch_shapes=[
            pltpu.VMEM((SQ, D), jnp.bfloat16),
            pltpu.VMEM((SQ, D), jnp.bfloat16),
            pltpu.SemaphoreType.DMA((18,)),
            pltpu.SemaphoreType.DMA((18,)),
        ],
        compiler_params=pltpu.CompilerParams(collective_id=0),
    )(x, Wq, Wo, Wk, Wv)


# device time: 32206 ns/iter; 1.0466x vs baseline; 1.0466x over previous
import os

import jax
import jax.numpy as jnp
from jax import lax
from jax.experimental import pallas as pl
from jax.experimental.pallas import tpu as pltpu

_VARIANT = os.environ.get("KERNEL_VARIANT", "full")

N_DEV = 8
SQ = 256
D = 1024
DH = 128
H_PER = 8
SCALE = 0.08838834764831843

RAIL_MASKS = ((1, 3, 4), (3, 4, 1), (4, 1, 3))
RAIL_COLS = ((0, 384), (384, 768), (768, 1024))
N_CHUNKS = 2
CHUNK_ROWS = 128
STAGE_OFF = (0, 64, 96)


def kernel(x, Wq, Wo, Wk, Wv):
    do_comm = _VARIANT != "compute"
    do_compute = _VARIANT != "comm"

    def body(x_ref, wq_ref, wo_ref, wk_ref, wv_ref, out_ref,
             acc_ref, rs_recv, send_sems, recv_sems):
        my = lax.axis_index("i")
        b0 = my & 1
        b1 = (my >> 1) & 1
        b2 = (my >> 2) & 1
        sels = (
            (b0 ^ b1, b1, b2),
            (b1, b2, b0),
            (b2, b0 ^ b1, b1),
        )

        if do_comm:
            barrier = pltpu.get_barrier_semaphore()
            for msk in (1, 3, 4):
                pl.semaphore_signal(barrier, inc=1, device_id=(my ^ msk,),
                                    device_id_type=pl.DeviceIdType.MESH)
            pl.semaphore_wait(barrier, 3)

        plans = []
        for c in range(N_CHUNKS):
            base = c * CHUNK_ROWS
            chunk_plans = []
            for r, ((m0, m1, m2), (s0, s1, s2)) in enumerate(
                    zip(RAIL_MASKS, sels)):
                k0 = base + s0 * 64
                k1 = k0 + s1 * 32
                k2 = k1 + s2 * 16
                chunk_plans.append({
                    "cols": RAIL_COLS[r],
                    "rs": [
                        (m0, base + (1 - s0) * 64, 64, k0),
                        (m1, k0 + (1 - s1) * 32, 32, k1),
                        (m2, k1 + (1 - s2) * 16, 16, k2),
                    ],
                    "ag": [(m2, k2, 16), (m1, k1, 32), (m0, k0, 64)],
                })
            plans.append(chunk_plans)

        def sem_idx(c, r, s):
            return (c * 3 + r) * 6 + s

        def rs_rdma(c, r, s):
            msk, src_row, sz, _ = plans[c][r]["rs"][s]
            c0, c1 = plans[c][r]["cols"]
            i = sem_idx(c, r, s)
            return pltpu.make_async_remote_copy(
                src_ref=acc_ref.at[pl.ds(src_row, sz), c0:c1],
                dst_ref=rs_recv.at[pl.ds(c * CHUNK_ROWS + STAGE_OFF[s], sz),
                                   c0:c1],
                send_sem=send_sems.at[i],
                recv_sem=recv_sems.at[i],
                device_id=(my ^ msk,),
                device_id_type=pl.DeviceIdType.MESH,
            )

        def ag_rdma(c, r, s):
            msk, row, sz = plans[c][r]["ag"][s]
            c0, c1 = plans[c][r]["cols"]
            i = sem_idx(c, r, 3 + s)
            return pltpu.make_async_remote_copy(
                src_ref=acc_ref.at[pl.ds(row, sz), c0:c1],
                dst_ref=acc_ref.at[pl.ds(row, sz), c0:c1],
                send_sem=send_sems.at[i],
                recv_sem=recv_sems.at[i],
                device_id=(my ^ msk,),
                device_id_type=pl.DeviceIdType.MESH,
            )

        bf = jnp.bfloat16
        if do_compute:
            xb = x_ref[0].astype(bf)
            wob = wo_ref[...].astype(bf)
            k = jnp.dot(xb, wk_ref[...].astype(bf),
                        preferred_element_type=jnp.float32).astype(bf)
            v = jnp.dot(xb, wv_ref[...].astype(bf),
                        preferred_element_type=jnp.float32).astype(bf)
            q = jnp.dot(xb, wq_ref[...].astype(bf),
                        preferred_element_type=jnp.float32).astype(bf)

            def compute_rows(r0):
                outs = []
                for h in range(H_PER):
                    sl = slice(h * DH, (h + 1) * DH)
                    s = jnp.dot(q[r0:r0 + 128, sl], k[:, sl].T,
                                preferred_element_type=jnp.float32) * SCALE
                    m = jnp.max(s, axis=-1, keepdims=True)
                    p = jnp.exp(s - m)
                    l = jnp.sum(p, axis=-1, keepdims=True)
                    outs.append(jnp.dot((p / l).astype(bf), v[:, sl],
                                        preferred_element_type=jnp.float32))
                attn = jnp.concatenate(outs, axis=1).astype(bf)
                part = jnp.dot(attn, wob,
                               preferred_element_type=jnp.float32)
                acc_ref[r0:r0 + 128, :] = part.astype(jnp.bfloat16)

            for c in range(N_CHUNKS):
                compute_rows(c * CHUNK_ROWS)
                if do_comm:
                    for r in range(3):
                        rs_rdma(c, r, 0).start()
        else:
            acc_ref[...] = x_ref[0].astype(bf)
            for c in range(N_CHUNKS):
                for r in range(3):
                    rs_rdma(c, r, 0).start()

        if do_comm:
            for s in range(3):
                for c in range(N_CHUNKS):
                    for r in range(3):
                        rs_rdma(c, r, s).wait()
                        _, _, sz, add_row = plans[c][r]["rs"][s]
                        cc0, cc1 = plans[c][r]["cols"]
                        if s < 2:
                            nxt = rs_rdma(c, r, s + 1)
                        else:
                            nxt = ag_rdma(c, r, 0)
                        acc_ref[pl.ds(add_row, sz), cc0:cc1] += (
                            rs_recv[pl.ds(c * CHUNK_ROWS + STAGE_OFF[s], sz),
                                    cc0:cc1])
                        nxt.start()
            for s in range(3):
                for c in range(N_CHUNKS):
                    for r in range(3):
                        ag_rdma(c, r, s).wait()
                        if s < 2:
                            ag_rdma(c, r, s + 1).start()

        out_ref[0] = acc_ref[...].astype(jnp.float32)

    return pl.pallas_call(
        body,
        out_shape=jax.ShapeDtypeStruct((1, SQ, D), jnp.float32),
        in_specs=[pl.BlockSpec(memory_space=pltpu.VMEM)] * 5,
        out_specs=pl.BlockSpec(memory_space=pltpu.VMEM),
        scratch_shapes=[
            pltpu.VMEM((SQ, D), jnp.bfloat16),
            pltpu.VMEM((SQ, D), jnp.bfloat16),
            pltpu.SemaphoreType.DMA((36,)),
            pltpu.SemaphoreType.DMA((36,)),
        ],
        compiler_params=(None if not do_comm
                         else pltpu.CompilerParams(collective_id=0)),
    )(x, Wq, Wo, Wk, Wv)


# device time: 30085 ns/iter; 1.1204x vs baseline; 1.0705x over previous
import os

import jax
import jax.numpy as jnp
from jax import lax
from jax.experimental import pallas as pl
from jax.experimental.pallas import tpu as pltpu

_VARIANT = os.environ.get("KERNEL_VARIANT", "full")
_TRANSPORT = os.environ.get("KERNEL_TRANSPORT", "bf16")

N_DEV = 8
SQ = 256
D = 1024
DH = 128
H_PER = 8
OWN = SQ // N_DEV
SCALE = 0.08838834764831843


def kernel(x, Wq, Wo, Wk, Wv):
    do_comm = _VARIANT != "compute"
    do_compute = _VARIANT != "comm"

    def body(x_ref, wq_ref, wo_ref, wk_ref, wv_ref, out_ref,
             acc_ref, rs_recv, send_sems, recv_sems):
        my = lax.axis_index("i")
        tdt = jnp.float32 if _TRANSPORT == "f32" else jnp.bfloat16

        if do_comm:
            barrier = pltpu.get_barrier_semaphore()
            for j in range(N_DEV - 1):
                d = (my + 1 + j) % N_DEV
                pl.semaphore_signal(barrier, inc=1, device_id=(d,),
                                    device_id_type=pl.DeviceIdType.MESH)
            pl.semaphore_wait(barrier, N_DEV - 1)

        def rs_rdma(j):
            d = (my + 1 + j) % N_DEV
            return pltpu.make_async_remote_copy(
                src_ref=acc_ref.at[pl.ds(d * OWN, OWN), :],
                dst_ref=rs_recv.at[pl.ds(j * OWN, OWN), :],
                send_sem=send_sems.at[j],
                recv_sem=recv_sems.at[j],
                device_id=(d,),
                device_id_type=pl.DeviceIdType.MESH,
            )

        def ag_rdma(j):
            d = (my + 1 + j) % N_DEV
            return pltpu.make_async_remote_copy(
                src_ref=acc_ref.at[pl.ds(my * OWN, OWN), :],
                dst_ref=acc_ref.at[pl.ds(my * OWN, OWN), :],
                send_sem=send_sems.at[N_DEV - 1 + j],
                recv_sem=recv_sems.at[N_DEV - 1 + j],
                device_id=(d,),
                device_id_type=pl.DeviceIdType.MESH,
            )

        bf = jnp.bfloat16
        if do_compute:
            xb = x_ref[0].astype(bf)
            wob = wo_ref[...].astype(bf)
            k = jnp.dot(xb, wk_ref[...].astype(bf),
                        preferred_element_type=jnp.float32).astype(bf)
            v = jnp.dot(xb, wv_ref[...].astype(bf),
                        preferred_element_type=jnp.float32).astype(bf)
            q = jnp.dot(xb, wq_ref[...].astype(bf),
                        preferred_element_type=jnp.float32).astype(bf)

            def compute_rows(r0):
                outs = []
                for h in range(H_PER):
                    sl = slice(h * DH, (h + 1) * DH)
                    s = jnp.dot(q[r0:r0 + 128, sl], k[:, sl].T,
                                preferred_element_type=jnp.float32) * SCALE
                    m = jnp.max(s, axis=-1, keepdims=True)
                    p = jnp.exp(s - m)
                    l = jnp.sum(p, axis=-1, keepdims=True)
                    outs.append(jnp.dot((p / l).astype(bf), v[:, sl],
                                        preferred_element_type=jnp.float32))
                attn = jnp.concatenate(outs, axis=1).astype(bf)
                part = jnp.dot(attn, wob,
                               preferred_element_type=jnp.float32)
                acc_ref[r0:r0 + 128, :] = part.astype(tdt)

            compute_rows(0)
            if do_comm:
                for j in range(N_DEV - 1):
                    d = (my + 1 + j) % N_DEV
                    @pl.when(d < N_DEV // 2)
                    def _(j=j):
                        rs_rdma(j).start()
            compute_rows(128)
            if do_comm:
                for j in range(N_DEV - 1):
                    d = (my + 1 + j) % N_DEV
                    @pl.when(d >= N_DEV // 2)
                    def _(j=j):
                        rs_rdma(j).start()
        else:
            acc_ref[...] = x_ref[0].astype(tdt)
            for j in range(N_DEV - 1):
                rs_rdma(j).start()

        if do_comm:
            for j in range(N_DEV - 1):
                rs_rdma(j).wait()
            t = acc_ref[pl.ds(my * OWN, OWN), :]
            for j in range(N_DEV - 1):
                t += rs_recv[j * OWN:(j + 1) * OWN, :]
            acc_ref[pl.ds(my * OWN, OWN), :] = t

            for j in range(N_DEV - 1):
                ag_rdma(j).start()
            for j in range(N_DEV - 1):
                ag_rdma(j).wait()

        out_ref[0] = acc_ref[...].astype(jnp.float32)

    tdt_s = jnp.float32 if _TRANSPORT == "f32" else jnp.bfloat16
    return pl.pallas_call(
        body,
        out_shape=jax.ShapeDtypeStruct((1, SQ, D), jnp.float32),
        in_specs=[pl.BlockSpec(memory_space=pltpu.VMEM)] * 5,
        out_specs=pl.BlockSpec(memory_space=pltpu.VMEM),
        scratch_shapes=[
            pltpu.VMEM((SQ, D), tdt_s),
            pltpu.VMEM(((N_DEV - 1) * OWN, D), tdt_s),
            pltpu.SemaphoreType.DMA((2 * (N_DEV - 1),)),
            pltpu.SemaphoreType.DMA((2 * (N_DEV - 1),)),
        ],
        compiler_params=(None if not do_comm
                         else pltpu.CompilerParams(collective_id=0)),
    )(x, Wq, Wo, Wk, Wv)


# device time: 29670 ns/iter; 1.1361x vs baseline; 1.0140x over previous
import os

import jax
import jax.numpy as jnp
from jax import lax
from jax.experimental import pallas as pl
from jax.experimental.pallas import tpu as pltpu

_VARIANT = os.environ.get("KERNEL_VARIANT", "full")
_TRANSPORT = os.environ.get("KERNEL_TRANSPORT", "bf16")

N_DEV = 8
SQ = 256
D = 1024
DH = 128
H_PER = 8
OWN = SQ // N_DEV
SCALE = 0.08838834764831843


def kernel(x, Wq, Wo, Wk, Wv):
    do_comm = _VARIANT != "compute"
    do_compute = _VARIANT != "comm"

    def body(x_ref, wq_ref, wo_ref, wk_ref, wv_ref, out_ref,
             acc_ref, rs_recv, send_sems, recv_sems):
        my = lax.axis_index("i")
        tdt = jnp.float32 if _TRANSPORT == "f32" else jnp.bfloat16

        if do_comm:
            barrier = pltpu.get_barrier_semaphore()
            for j in range(N_DEV - 1):
                d = (my + 1 + j) % N_DEV
                pl.semaphore_signal(barrier, inc=1, device_id=(d,),
                                    device_id_type=pl.DeviceIdType.MESH)
            pl.semaphore_wait(barrier, N_DEV - 1)

        def rs_rdma(j):
            d = (my + 1 + j) % N_DEV
            return pltpu.make_async_remote_copy(
                src_ref=acc_ref.at[pl.ds(d * OWN, OWN), :],
                dst_ref=rs_recv.at[pl.ds(j * OWN, OWN), :],
                send_sem=send_sems.at[j],
                recv_sem=recv_sems.at[j],
                device_id=(d,),
                device_id_type=pl.DeviceIdType.MESH,
            )

        def ag_rdma(j):
            d = (my + 1 + j) % N_DEV
            return pltpu.make_async_remote_copy(
                src_ref=acc_ref.at[pl.ds(my * OWN, OWN), :],
                dst_ref=acc_ref.at[pl.ds(my * OWN, OWN), :],
                send_sem=send_sems.at[N_DEV - 1 + j],
                recv_sem=recv_sems.at[N_DEV - 1 + j],
                device_id=(d,),
                device_id_type=pl.DeviceIdType.MESH,
            )

        bf = jnp.bfloat16
        if do_compute:
            xb = x_ref[0].astype(bf)
            k = jnp.dot(xb, wk_ref[...].astype(bf),
                        preferred_element_type=jnp.float32).astype(bf)
            v = jnp.dot(xb, wv_ref[...].astype(bf),
                        preferred_element_type=jnp.float32).astype(bf)
            q = jnp.dot(xb, wq_ref[...].astype(bf),
                        preferred_element_type=jnp.float32).astype(bf)
            wob = wo_ref[...].astype(bf)

            def compute_rows(r0):
                outs = []
                for h in range(H_PER):
                    sl = slice(h * DH, (h + 1) * DH)
                    s = jnp.dot(q[r0:r0 + 128, sl], k[:, sl].T,
                                preferred_element_type=jnp.float32) * SCALE
                    m = jnp.max(s, axis=-1, keepdims=True)
                    p = jnp.exp(s - m)
                    l = jnp.sum(p, axis=-1, keepdims=True)
                    o = jnp.dot(p.astype(bf), v[:, sl],
                                preferred_element_type=jnp.float32)
                    outs.append(o / l)
                attn = jnp.concatenate(outs, axis=1).astype(bf)
                part = jnp.dot(attn, wob,
                               preferred_element_type=jnp.float32)
                acc_ref[r0:r0 + 128, :] = part.astype(tdt)

            compute_rows(0)
            if do_comm:
                for j in range(N_DEV - 1):
                    d = (my + 1 + j) % N_DEV
                    @pl.when(d < N_DEV // 2)
                    def _(j=j):
                        rs_rdma(j).start()
            compute_rows(128)
            if do_comm:
                for j in range(N_DEV - 1):
                    d = (my + 1 + j) % N_DEV
                    @pl.when(d >= N_DEV // 2)
                    def _(j=j):
                        rs_rdma(j).start()
        else:
            acc_ref[...] = x_ref[0].astype(tdt)
            for j in range(N_DEV - 1):
                rs_rdma(j).start()

        if do_comm:
            for j in range(N_DEV - 1):
                rs_rdma(j).wait()
            t = acc_ref[pl.ds(my * OWN, OWN), :]
            for j in range(N_DEV - 1):
                t += rs_recv[j * OWN:(j + 1) * OWN, :]
            acc_ref[pl.ds(my * OWN, OWN), :] = t

            for j in range(N_DEV - 1):
                ag_rdma(j).start()
            out_ref[0, pl.ds(my * OWN, OWN), :] = t.astype(jnp.float32)
            for j in range(N_DEV - 1):
                ag_rdma(j).wait()
                s_dev = (my - 1 - j) % N_DEV
                out_ref[0, pl.ds(s_dev * OWN, OWN), :] = (
                    acc_ref[pl.ds(s_dev * OWN, OWN), :].astype(jnp.float32))
        else:
            out_ref[0] = acc_ref[...].astype(jnp.float32)

    tdt_s = jnp.float32 if _TRANSPORT == "f32" else jnp.bfloat16
    return pl.pallas_call(
        body,
        out_shape=jax.ShapeDtypeStruct((1, SQ, D), jnp.float32),
        in_specs=[pl.BlockSpec(memory_space=pltpu.VMEM)] * 5,
        out_specs=pl.BlockSpec(memory_space=pltpu.VMEM),
        scratch_shapes=[
            pltpu.VMEM((SQ, D), tdt_s),
            pltpu.VMEM(((N_DEV - 1) * OWN, D), tdt_s),
            pltpu.SemaphoreType.DMA((2 * (N_DEV - 1),)),
            pltpu.SemaphoreType.DMA((2 * (N_DEV - 1),)),
        ],
        compiler_params=(None if not do_comm
                         else pltpu.CompilerParams(collective_id=0)),
    )(x, Wq, Wo, Wk, Wv)
